# Initial kernel scaffold; baseline (speedup 1.0000x reference)
import jax
import jax.numpy as jnp
from jax.experimental import pallas as pl


def kernel(input_sequence, hidden, table, W_ih, W_hh):
    raise NotImplementedError("write your pallas kernel here")


# speedup vs baseline: 10.7614x; 10.7614x over previous
import jax
import jax.numpy as jnp
from jax.experimental import pallas as pl
from jax.experimental.pallas import tpu as pltpu
from jax.experimental.pallas import tpu_sc as plsc


_GATHER_WINDOW = 128


def _sc_gather(table, flat_idx):
    n = flat_idx.shape[1]
    e = table.shape[1]
    mesh = plsc.VectorSubcoreMesh(core_axis_name="core", subcore_axis_name="subcore")

    @pl.kernel(
        out_type=jax.ShapeDtypeStruct((n, e), table.dtype),
        mesh=mesh,
    )
    def gather_kernel(tab_hbm, idx_hbm, out_hbm):
        def body(idx_vmem, out_vmem):
            pltpu.sync_copy(tab_hbm.at[idx_vmem.at[0]], out_vmem)

        pltpu.emit_pipeline(
            body,
            grid=(n // _GATHER_WINDOW,),
            in_specs=[
                pl.BlockSpec((1, _GATHER_WINDOW), index_map=lambda i: (0, i))
            ],
            out_specs=[
                pl.BlockSpec((_GATHER_WINDOW, e), index_map=lambda i: (i, 0))
            ],
            core_axis_name=("core", "subcore"),
            dimension_semantics=(pltpu.PARALLEL,),
        )(idx_hbm, out_hbm)

    return gather_kernel(table, flat_idx)


def _gru_scan(emb_lbe, w_ih_t, w_hh_t):
    l, b, e = emb_lbe.shape
    h_dim = w_hh_t.shape[0]

    def body(emb_ref, wih_ref, whh_ref, out_ref, hlast_ref):
        t = pl.program_id(0)

        @pl.when(t == 0)
        def _():
            hlast_ref[...] = jnp.zeros_like(hlast_ref)

        x = emb_ref[0]
        h = hlast_ref[...]
        gi = jnp.dot(x, wih_ref[...], preferred_element_type=jnp.float32)
        gh = jnp.dot(h, whh_ref[...], preferred_element_type=jnp.float32)
        r = jax.nn.sigmoid(gi[:, :h_dim] + gh[:, :h_dim])
        z = jax.nn.sigmoid(gi[:, h_dim:2 * h_dim] + gh[:, h_dim:2 * h_dim])
        n = jnp.tanh(gi[:, 2 * h_dim:] + r * gh[:, 2 * h_dim:])
        h_new = (1.0 - z) * n + z * h
        hlast_ref[...] = h_new
        out_ref[...] = h_new

    out_flat, h_last = pl.pallas_call(
        body,
        grid=(l,),
        in_specs=[
            pl.BlockSpec((1, b, e), lambda t: (t, 0, 0)),
            pl.BlockSpec((e, 3 * h_dim), lambda t: (0, 0)),
            pl.BlockSpec((h_dim, 3 * h_dim), lambda t: (0, 0)),
        ],
        out_specs=[
            pl.BlockSpec((b, h_dim), lambda t: (0, t)),
            pl.BlockSpec((b, h_dim), lambda t: (0, 0)),
        ],
        out_shape=[
            jax.ShapeDtypeStruct((b, l * h_dim), jnp.float32),
            jax.ShapeDtypeStruct((b, h_dim), jnp.float32),
        ],
    )(emb_lbe, w_ih_t, w_hh_t)
    return out_flat, h_last


def kernel(input_sequence, hidden, table, W_ih, W_hh):
    del hidden
    b, l = input_sequence.shape
    h_dim = W_hh.shape[1]
    idx = input_sequence.astype(jnp.int32).T.reshape(1, l * b)
    emb = _sc_gather(table, idx).reshape(l, b, table.shape[1])
    out_flat, h_last = _gru_scan(emb, W_ih.T, W_hh.T)
    return out_flat.reshape(b, l, h_dim), h_last[None]


# speedup vs baseline: 11.1636x; 1.0374x over previous
import jax
import jax.numpy as jnp
from jax.experimental import pallas as pl
from jax.experimental.pallas import tpu as pltpu
from jax.experimental.pallas import tpu_sc as plsc


_GATHER_WINDOW = 128


def _sc_gather(table, flat_idx):
    n = flat_idx.shape[1]
    e = table.shape[1]
    mesh = plsc.VectorSubcoreMesh(core_axis_name="core", subcore_axis_name="subcore")

    @pl.kernel(
        out_type=jax.ShapeDtypeStruct((n, e), table.dtype),
        mesh=mesh,
    )
    def gather_kernel(tab_hbm, idx_hbm, out_hbm):
        def body(idx_vmem, out_vmem):
            pltpu.sync_copy(tab_hbm.at[idx_vmem.at[0]], out_vmem)

        pltpu.emit_pipeline(
            body,
            grid=(n // _GATHER_WINDOW,),
            in_specs=[
                pl.BlockSpec((1, _GATHER_WINDOW), index_map=lambda i: (0, i))
            ],
            out_specs=[
                pl.BlockSpec((_GATHER_WINDOW, e), index_map=lambda i: (i, 0))
            ],
            core_axis_name=("core", "subcore"),
            dimension_semantics=(pltpu.PARALLEL,),
        )(idx_hbm, out_hbm)

    return gather_kernel(table, flat_idx)


_STEPS_PER_ITER = 8


def _gru_scan(emb_flat, w_ih_t, w_hh_t, l):
    b = emb_flat.shape[0]
    e = w_ih_t.shape[0]
    h_dim = w_hh_t.shape[0]
    t_blk = _STEPS_PER_ITER

    def body(emb_ref, wih_ref, whh_ref, out_ref, hlast_ref):
        i = pl.program_id(0)

        @pl.when(i == 0)
        def _():
            hlast_ref[...] = jnp.zeros_like(hlast_ref)

        h = hlast_ref[...]
        wih = wih_ref[...]
        whh = whh_ref[...]
        for t in range(t_blk):
            x = emb_ref[:, t * e:(t + 1) * e]
            gi = jnp.dot(x, wih, preferred_element_type=jnp.float32)
            gh = jnp.dot(h, whh, preferred_element_type=jnp.float32)
            r = jax.nn.sigmoid(gi[:, :h_dim] + gh[:, :h_dim])
            z = jax.nn.sigmoid(gi[:, h_dim:2 * h_dim] + gh[:, h_dim:2 * h_dim])
            n = jnp.tanh(gi[:, 2 * h_dim:] + r * gh[:, 2 * h_dim:])
            h = (1.0 - z) * n + z * h
            out_ref[:, t * h_dim:(t + 1) * h_dim] = h
        hlast_ref[...] = h

    out_flat, h_last = pl.pallas_call(
        body,
        grid=(l // t_blk,),
        in_specs=[
            pl.BlockSpec((b, t_blk * e), lambda i: (0, i)),
            pl.BlockSpec((e, 3 * h_dim), lambda i: (0, 0)),
            pl.BlockSpec((h_dim, 3 * h_dim), lambda i: (0, 0)),
        ],
        out_specs=[
            pl.BlockSpec((b, t_blk * h_dim), lambda i: (0, i)),
            pl.BlockSpec((b, h_dim), lambda i: (0, 0)),
        ],
        out_shape=[
            jax.ShapeDtypeStruct((b, l * h_dim), jnp.float32),
            jax.ShapeDtypeStruct((b, h_dim), jnp.float32),
        ],
    )(emb_flat, w_ih_t, w_hh_t)
    return out_flat, h_last


def kernel(input_sequence, hidden, table, W_ih, W_hh):
    del hidden
    b, l = input_sequence.shape
    e = table.shape[1]
    h_dim = W_hh.shape[1]
    idx = input_sequence.astype(jnp.int32).reshape(1, b * l)
    emb_flat = _sc_gather(table, idx).reshape(b, l * e)
    out_flat, h_last = _gru_scan(emb_flat, W_ih.T, W_hh.T, l)
    return out_flat.reshape(b, l, h_dim), h_last[None]


# speedup vs baseline: 14.5184x; 1.3005x over previous
import jax
import jax.numpy as jnp
from jax.experimental import pallas as pl
from jax.experimental.pallas import tpu as pltpu
from jax.experimental.pallas import tpu_sc as plsc


_GATHER_WINDOW = 128


def _sc_gather(table, flat_idx):
    n = flat_idx.shape[1]
    e = table.shape[1]
    mesh = plsc.VectorSubcoreMesh(core_axis_name="core", subcore_axis_name="subcore")

    @pl.kernel(
        out_type=jax.ShapeDtypeStruct((n, e), table.dtype),
        mesh=mesh,
    )
    def gather_kernel(tab_hbm, idx_hbm, out_hbm):
        def body(idx_vmem, out_vmem):
            pltpu.sync_copy(tab_hbm.at[idx_vmem.at[0]], out_vmem)

        pltpu.emit_pipeline(
            body,
            grid=(n // _GATHER_WINDOW,),
            in_specs=[
                pl.BlockSpec((1, _GATHER_WINDOW), index_map=lambda i: (0, i))
            ],
            out_specs=[
                pl.BlockSpec((_GATHER_WINDOW, e), index_map=lambda i: (i, 0))
            ],
            core_axis_name=("core", "subcore"),
            dimension_semantics=(pltpu.PARALLEL,),
        )(idx_hbm, out_hbm)

    return gather_kernel(table, flat_idx)


_STEPS_PER_ITER = 8


def _gru_scan(emb, w_ih_t, w_hh_t):
    b, l, e = emb.shape
    h_dim = w_hh_t.shape[0]
    t_blk = _STEPS_PER_ITER

    def body(emb_ref, wih_ref, whh_ref, out_ref, hlast_ref):
        i = pl.program_id(0)

        @pl.when(i == 0)
        def _():
            hlast_ref[...] = jnp.zeros_like(hlast_ref)

        h = hlast_ref[...]
        wih = wih_ref[...]
        whh = whh_ref[...]
        for t in range(t_blk):
            x = emb_ref[:, t, :]
            gi = jnp.dot(x, wih, preferred_element_type=jnp.float32)
            gh = jnp.dot(h, whh, preferred_element_type=jnp.float32)
            r = jax.nn.sigmoid(gi[:, :h_dim] + gh[:, :h_dim])
            z = jax.nn.sigmoid(gi[:, h_dim:2 * h_dim] + gh[:, h_dim:2 * h_dim])
            n = jnp.tanh(gi[:, 2 * h_dim:] + r * gh[:, 2 * h_dim:])
            h = (1.0 - z) * n + z * h
            out_ref[:, t, :] = h
        hlast_ref[...] = h

    out, h_last = pl.pallas_call(
        body,
        grid=(l // t_blk,),
        in_specs=[
            pl.BlockSpec((b, t_blk, e), lambda i: (0, i, 0)),
            pl.BlockSpec((e, 3 * h_dim), lambda i: (0, 0)),
            pl.BlockSpec((h_dim, 3 * h_dim), lambda i: (0, 0)),
        ],
        out_specs=[
            pl.BlockSpec((b, t_blk, h_dim), lambda i: (0, i, 0)),
            pl.BlockSpec((b, h_dim), lambda i: (0, 0)),
        ],
        out_shape=[
            jax.ShapeDtypeStruct((b, l, h_dim), jnp.float32),
            jax.ShapeDtypeStruct((b, h_dim), jnp.float32),
        ],
    )(emb, w_ih_t, w_hh_t)
    return out, h_last


def kernel(input_sequence, hidden, table, W_ih, W_hh):
    del hidden
    b, l = input_sequence.shape
    e = table.shape[1]
    idx = input_sequence.astype(jnp.int32).reshape(1, b * l)
    emb = _sc_gather(table, idx).reshape(b, l, e)
    out, h_last = _gru_scan(emb, W_ih.T, W_hh.T)
    return out, h_last[None]


# speedup vs baseline: 16.4126x; 1.1305x over previous
import jax
import jax.numpy as jnp
from jax.experimental import pallas as pl
from jax.experimental.pallas import tpu as pltpu
from jax.experimental.pallas import tpu_sc as plsc


_GATHER_WINDOW = 128


def _sc_gather(table, flat_idx):
    n = flat_idx.shape[1]
    e = table.shape[1]
    mesh = plsc.VectorSubcoreMesh(core_axis_name="core", subcore_axis_name="subcore")

    @pl.kernel(
        out_type=jax.ShapeDtypeStruct((n, e), table.dtype),
        mesh=mesh,
    )
    def gather_kernel(tab_hbm, idx_hbm, out_hbm):
        def body(idx_vmem, out_vmem):
            pltpu.sync_copy(tab_hbm.at[idx_vmem.at[0]], out_vmem)

        pltpu.emit_pipeline(
            body,
            grid=(n // _GATHER_WINDOW,),
            in_specs=[
                pl.BlockSpec((1, _GATHER_WINDOW), index_map=lambda i: (0, i))
            ],
            out_specs=[
                pl.BlockSpec((_GATHER_WINDOW, e), index_map=lambda i: (i, 0))
            ],
            core_axis_name=("core", "subcore"),
            dimension_semantics=(pltpu.PARALLEL,),
        )(idx_hbm, out_hbm)

    return gather_kernel(table, flat_idx)


_STEPS_PER_ITER = 8


def _gru_scan(emb_tmaj, w_ih_t, w_hh_t, b):
    n_chunks, tb, e = emb_tmaj.shape
    h_dim = w_hh_t.shape[0]
    t_blk = _STEPS_PER_ITER
    l = n_chunks * t_blk

    def body(emb_ref, wih_ref, whh_ref, out_ref, hlast_ref):
        i = pl.program_id(0)

        @pl.when(i == 0)
        def _():
            hlast_ref[...] = jnp.zeros_like(hlast_ref)

        h = hlast_ref[...]
        wih = wih_ref[...]
        whh = whh_ref[...]
        for t in range(t_blk):
            x = emb_ref[0, t * b:(t + 1) * b, :]
            gi = jnp.dot(x, wih, preferred_element_type=jnp.float32)
            gh = jnp.dot(h, whh, preferred_element_type=jnp.float32)
            r = 0.5 * jnp.tanh(0.5 * (gi[:, :h_dim] + gh[:, :h_dim])) + 0.5
            z = 0.5 * jnp.tanh(0.5 * (gi[:, h_dim:2 * h_dim] + gh[:, h_dim:2 * h_dim])) + 0.5
            n = jnp.tanh(gi[:, 2 * h_dim:] + r * gh[:, 2 * h_dim:])
            h = n + z * (h - n)
            out_ref[:, t, :] = h
        hlast_ref[...] = h

    out, h_last = pl.pallas_call(
        body,
        grid=(n_chunks,),
        in_specs=[
            pl.BlockSpec((1, t_blk * b, e), lambda i: (i, 0, 0)),
            pl.BlockSpec((e, 3 * h_dim), lambda i: (0, 0)),
            pl.BlockSpec((h_dim, 3 * h_dim), lambda i: (0, 0)),
        ],
        out_specs=[
            pl.BlockSpec((b, t_blk, h_dim), lambda i: (0, i, 0)),
            pl.BlockSpec((b, h_dim), lambda i: (0, 0)),
        ],
        out_shape=[
            jax.ShapeDtypeStruct((b, l, h_dim), jnp.float32),
            jax.ShapeDtypeStruct((b, h_dim), jnp.float32),
        ],
    )(emb_tmaj, w_ih_t, w_hh_t)
    return out, h_last


def kernel(input_sequence, hidden, table, W_ih, W_hh):
    del hidden
    b, l = input_sequence.shape
    e = table.shape[1]
    t_blk = _STEPS_PER_ITER
    idx = input_sequence.astype(jnp.int32).T.reshape(1, b * l)
    emb_tmaj = _sc_gather(table, idx).reshape(l // t_blk, t_blk * b, e)
    out, h_last = _gru_scan(emb_tmaj, W_ih.T, W_hh.T, b)
    return out, h_last[None]
